# baseline (speedup 1.0000x reference)
import functools

import jax
import jax.numpy as jnp
from jax import lax
from jax.experimental import pallas as pl
from jax.experimental.pallas import tpu as pltpu
from jax.experimental.pallas import tpu_sc as plsc

B = 4096
N_CAT = 26
N_CAT_PAD = 32
N_NUM = 16
N_TOK = N_CAT + N_NUM
D = 32
VOCAB = 100000

NC = 2
NS = 16
NW = NC * NS
B_PER_W = B // NW
CHUNK = 64
N_CHUNKS = B_PER_W // CHUNK
GATHER_GROUP = 16


def _sc_body(x_cat_hbm, x_num_hbm, table_hbm, w_hbm, bias_hbm, out_hbm,
             idx_v, xnum_v, w_v, bias_v, outblk_v, sem):
    cid = lax.axis_index("c")
    sid = lax.axis_index("s")
    wid = sid * NC + cid
    base0 = wid * B_PER_W

    pltpu.sync_copy(w_hbm, w_v)
    pltpu.sync_copy(bias_hbm, bias_v)

    i16 = lax.iota(jnp.int32, 16)
    off_lo = i16 * VOCAB
    off_hi = jnp.where(i16 < N_CAT - 16, (i16 + 16) * VOCAB, 0)

    for c in range(N_CHUNKS):
        base = base0 + c * CHUNK
        pltpu.sync_copy(x_cat_hbm.at[pl.ds(base, CHUNK)], idx_v)
        pltpu.sync_copy(x_num_hbm.at[pl.ds(base, CHUNK)], xnum_v)

        def idx_body(b, _):
            idx_v[b, pl.ds(0, 16)] = idx_v[b, pl.ds(0, 16)] + off_lo
            idx_v[b, pl.ds(16, 16)] = idx_v[b, pl.ds(16, 16)] + off_hi
            return _
        lax.fori_loop(0, CHUNK, idx_body, None)

        def gather_group(g, _):
            copies = []
            for i in range(GATHER_GROUP):
                b = g * GATHER_GROUP + i
                copies.append(pltpu.async_copy(
                    table_hbm.at[idx_v.at[b]],
                    outblk_v.at[b, pl.ds(0, N_CAT_PAD)],
                    sem))
            for cp in copies:
                cp.wait()
            return _
        lax.fori_loop(0, CHUNK // GATHER_GROUP, gather_group, None)

        def num_body(b, _):
            xrow = xnum_v[b, pl.ds(0, N_NUM)]
            for n in range(N_NUM):
                xv = jnp.full((16,), xrow[n], jnp.float32)
                for h in range(2):
                    s = pl.ds(h * 16, 16)
                    outblk_v[b, N_CAT + n, s] = (
                        xv * w_v[n, s] + bias_v[n, s])
            return _
        lax.fori_loop(0, CHUNK, num_body, None)

        pltpu.sync_copy(outblk_v, out_hbm.at[pl.ds(base, CHUNK)])


@jax.jit
def kernel(x_cat, x_num, cat_tables, num_weight, num_bias):
    x_cat_p = jnp.pad(x_cat, ((0, 0), (0, N_CAT_PAD - N_CAT)))
    table_flat = cat_tables.reshape(N_CAT * VOCAB, D)

    mesh = plsc.VectorSubcoreMesh(core_axis_name="c", subcore_axis_name="s")
    run = pl.kernel(
        _sc_body,
        out_type=jax.ShapeDtypeStruct((B, N_TOK, D), jnp.float32),
        mesh=mesh,
        compiler_params=pltpu.CompilerParams(use_tc_tiling_on_sc=False),
        scratch_types=[
            pltpu.VMEM((CHUNK, N_CAT_PAD), jnp.int32),
            pltpu.VMEM((CHUNK, N_NUM), jnp.float32),
            pltpu.VMEM((N_NUM, D), jnp.float32),
            pltpu.VMEM((N_NUM, D), jnp.float32),
            pltpu.VMEM((CHUNK, N_TOK, D), jnp.float32),
            pltpu.SemaphoreType.DMA,
        ],
    )
    return run(x_cat_p, x_num, table_flat, num_weight, num_bias)


# speedup vs baseline: 7.1753x; 7.1753x over previous
import functools

import jax
import jax.numpy as jnp
from jax import lax
from jax.experimental import pallas as pl
from jax.experimental.pallas import tpu as pltpu
from jax.experimental.pallas import tpu_sc as plsc

B = 4096
N_CAT = 26
N_NUM = 16
N_TOK = N_CAT + N_NUM
D = 32
VOCAB = 100000

NC = 2
NS = 16
NW = NC * NS
ROWS = N_CAT * D
ROWS_PER_W = ROWS // NW
L = 16
NUM_SEG = 512


def _sc_body(tab_hbm, idx_hbm, xnum_hbm, w_hbm, bias_hbm, out_hbm,
             rowbuf_v, idx_v, outrow_v, numblk_v, xn_v, wb_v,
             sem_row, sem_idx, sem_out):
    cid = lax.axis_index("c")
    sid = lax.axis_index("s")
    wid = sid * NC + cid

    pltpu.sync_copy(w_hbm, wb_v.at[0])
    pltpu.sync_copy(bias_hbm, wb_v.at[1])

    n = wid // 2
    dh = wid % 2
    for lseg in range(B // NUM_SEG):
        pltpu.sync_copy(xnum_hbm.at[n, pl.ds(lseg * NUM_SEG, NUM_SEG)], xn_v)
        for dj in range(D // 2):
            wrow = wb_v[0, n, pl.ds(dh * L, L)]
            brow = wb_v[1, n, pl.ds(dh * L, L)]
            ws = wrow[dj]
            bs = brow[dj]
            for c in range(NUM_SEG // L):
                x = xn_v[pl.ds(c * L, L)]
                numblk_v[dj, pl.ds(c * L, L)] = x * ws + bs
        pltpu.async_copy(
            numblk_v,
            out_hbm.at[N_CAT + n, pl.ds(dh * (D // 2), D // 2),
                       pl.ds(lseg * NUM_SEG, NUM_SEG)],
            sem_out).wait()

    for j in range(ROWS_PER_W):
        r = wid * ROWS_PER_W + j
        f = r // D
        d = r % D
        row_cp = pltpu.async_copy(tab_hbm.at[r], rowbuf_v, sem_row)
        idx_cp = pltpu.async_copy(idx_hbm.at[f], idx_v, sem_idx)
        idx_cp.wait()
        row_cp.wait()

        def gather_body(k, _):
            iv = idx_v[pl.ds(k * L, L)]
            outrow_v[pl.ds(k * L, L)] = plsc.load_gather(rowbuf_v, [iv])
            return _
        lax.fori_loop(0, B // L, gather_body, None)

        pltpu.async_copy(outrow_v, out_hbm.at[f, d], sem_out).wait()


@jax.jit
def kernel(x_cat, x_num, cat_tables, num_weight, num_bias):
    tab_t = cat_tables.transpose(0, 2, 1).reshape(ROWS, VOCAB)
    idx_t = x_cat.T
    xnum_t = x_num.T

    mesh = plsc.VectorSubcoreMesh(core_axis_name="c", subcore_axis_name="s")
    run = pl.kernel(
        _sc_body,
        out_type=jax.ShapeDtypeStruct((N_TOK, D, B), jnp.float32),
        mesh=mesh,
        compiler_params=pltpu.CompilerParams(needs_layout_passes=False),
        scratch_types=[
            pltpu.VMEM((VOCAB,), jnp.float32),
            pltpu.VMEM((B,), jnp.int32),
            pltpu.VMEM((B,), jnp.float32),
            pltpu.VMEM((D // 2, NUM_SEG), jnp.float32),
            pltpu.VMEM((NUM_SEG,), jnp.float32),
            pltpu.VMEM((2, N_NUM, D), jnp.float32),
            pltpu.SemaphoreType.DMA,
            pltpu.SemaphoreType.DMA,
            pltpu.SemaphoreType.DMA,
        ],
    )
    out_t = run(tab_t, idx_t, xnum_t, num_weight, num_bias)
    return out_t.transpose(2, 0, 1)
